# baseline (speedup 1.0000x reference)
import functools

import jax
import jax.numpy as jnp
from jax import lax
from jax.experimental import pallas as pl
from jax.experimental.pallas import tpu as pltpu
from jax.experimental.pallas import tpu_sc as plsc

_D = 128
_NC = 2
_NS = 16
_NW = _NC * _NS


def _sc_gather_pair(Wu, Wi, x_u, x_i, start, rows):
    b_per_w = rows // _NW
    mesh = plsc.VectorSubcoreMesh(core_axis_name="c", subcore_axis_name="s")

    ch = min(256, b_per_w)
    n_ch = b_per_w // ch

    @functools.partial(
        pl.kernel,
        mesh=mesh,
        out_type=(
            jax.ShapeDtypeStruct((rows, _D), jnp.float32),
            jax.ShapeDtypeStruct((rows, _D), jnp.float32),
        ),
        scratch_types=[
            pltpu.VMEM((b_per_w,), jnp.int32),
            pltpu.VMEM((b_per_w,), jnp.int32),
            pltpu.VMEM((ch, _D), jnp.float32),
            pltpu.VMEM((ch, _D), jnp.float32),
            pltpu.SemaphoreType.DMA,
            pltpu.SemaphoreType.DMA,
            pltpu.SemaphoreType.DMA,
            pltpu.SemaphoreType.DMA,
        ],
    )
    def gather_kernel(wu_hbm, wi_hbm, xu_hbm, xi_hbm, ou_hbm, oi_hbm,
                      idxu_v, idxi_v, rows0, rows1, g0, g1, w0, w1):
        wid = lax.axis_index("s") * _NC + lax.axis_index("c")
        base = wid * b_per_w
        pltpu.sync_copy(xu_hbm.at[pl.ds(start + base, b_per_w)], idxu_v)
        pltpu.sync_copy(xi_hbm.at[pl.ds(start + base, b_per_w)], idxi_v)

        bufs = (rows0, rows1)
        gsem = (g0, g1)
        wsem = (w0, w1)
        work = [(idxu_v, wu_hbm, ou_hbm, k) for k in range(n_ch)]
        work += [(idxi_v, wi_hbm, oi_hbm, k) for k in range(n_ch)]

        gathers = [None] * len(work)
        writes = [None] * len(work)
        for j, (idx_v, tbl, out, k) in enumerate(work):
            b = j % 2
            if j >= 2:
                writes[j - 2].wait()
            gathers[j] = pltpu.async_copy(
                tbl.at[idx_v.at[pl.ds(k * ch, ch)]], bufs[b], gsem[b])
            if j >= 1:
                pj, (_, _, pout, pk) = j - 1, work[j - 1]
                gathers[pj].wait()
                writes[pj] = pltpu.async_copy(
                    bufs[pj % 2], pout.at[pl.ds(base + pk * ch, ch)],
                    wsem[pj % 2])
        last = len(work) - 1
        _, _, lout, lk = work[last]
        gathers[last].wait()
        writes[last] = pltpu.async_copy(
            bufs[last % 2], lout.at[pl.ds(base + lk * ch, ch)],
            wsem[last % 2])
        writes[last - 1].wait()
        writes[last].wait()

    return gather_kernel(Wu, Wi, x_u, x_i)


def _mlp_body(hu_ref, hi_ref, w1_ref, b1_ref, w2_ref, b2_ref, wo_ref, bo_ref,
              o_ref):
    w1 = w1_ref[...]
    dn = (((1,), (1,)), ((), ()))
    a = lax.dot_general(hu_ref[...], w1[:, :_D], dn,
                        preferred_element_type=jnp.float32)
    a = a + lax.dot_general(hi_ref[...], w1[:, _D:], dn,
                            preferred_element_type=jnp.float32)
    a = a + b1_ref[...]
    a = jnp.where(a >= 0, a, 0.01 * a)
    b = lax.dot_general(a, w2_ref[...], dn,
                        preferred_element_type=jnp.float32)
    b = b + b2_ref[...]
    b = jnp.where(b >= 0, b, 0.01 * b)
    o_ref[...] = lax.dot_general(wo_ref[...], b, dn,
                                 preferred_element_type=jnp.float32) + bo_ref[...]


def _mlp(hu, hi, W1, b1, W2, b2, Wo, bo, block_rows=1024):
    B = hu.shape[0]
    return pl.pallas_call(
        _mlp_body,
        grid=(B // block_rows,),
        in_specs=[
            pl.BlockSpec((block_rows, _D), lambda i: (i, 0)),
            pl.BlockSpec((block_rows, _D), lambda i: (i, 0)),
            pl.BlockSpec((2 * _D, 2 * _D), lambda i: (0, 0)),
            pl.BlockSpec((1, 2 * _D), lambda i: (0, 0)),
            pl.BlockSpec((2 * _D, 2 * _D), lambda i: (0, 0)),
            pl.BlockSpec((1, 2 * _D), lambda i: (0, 0)),
            pl.BlockSpec((1, 2 * _D), lambda i: (0, 0)),
            pl.BlockSpec((1, 1), lambda i: (0, 0)),
        ],
        out_specs=pl.BlockSpec((1, block_rows), lambda i: (0, i)),
        out_shape=jax.ShapeDtypeStruct((1, B), jnp.float32),
    )(hu, hi, W1, b1.reshape(1, -1), W2, b2.reshape(1, -1), Wo,
      bo.reshape(1, 1))


def kernel(x_u, x_i, Wu, Wi, W1, b1, W2, b2, Wo, bo):
    B = x_u.shape[0]
    x_u = x_u.astype(jnp.int32)
    x_i = x_i.astype(jnp.int32)
    h = B // 2
    g0 = _sc_gather_pair(Wu, Wi, x_u, x_i, 0, h)
    g1 = _sc_gather_pair(Wu, Wi, x_u, x_i, h, B - h)
    o0 = _mlp(g0[0], g0[1], W1, b1, W2, b2, Wo, bo, block_rows=2048)
    o1 = _mlp(g1[0], g1[1], W1, b1, W2, b2, Wo, bo, block_rows=2048)
    return jnp.concatenate([o0, o1], axis=1).reshape(B, 1)


# speedup vs baseline: 1.0277x; 1.0277x over previous
import functools

import jax
import jax.numpy as jnp
from jax import lax
from jax.experimental import pallas as pl
from jax.experimental.pallas import tpu as pltpu
from jax.experimental.pallas import tpu_sc as plsc

_D = 128
_NC = 2
_NS = 16
_NW = _NC * _NS


def _sc_gather_pair(Wu, Wi, x_u, x_i, start, rows):
    b_per_w = rows // _NW
    mesh = plsc.VectorSubcoreMesh(core_axis_name="c", subcore_axis_name="s")

    ch = min(256, b_per_w)
    n_ch = b_per_w // ch

    @functools.partial(
        pl.kernel,
        mesh=mesh,
        out_type=(
            jax.ShapeDtypeStruct((rows, _D), jnp.float32),
            jax.ShapeDtypeStruct((rows, _D), jnp.float32),
        ),
        scratch_types=[
            pltpu.VMEM((b_per_w,), jnp.int32),
            pltpu.VMEM((b_per_w,), jnp.int32),
            pltpu.VMEM((ch, _D), jnp.float32),
            pltpu.VMEM((ch, _D), jnp.float32),
            pltpu.SemaphoreType.DMA,
            pltpu.SemaphoreType.DMA,
            pltpu.SemaphoreType.DMA,
            pltpu.SemaphoreType.DMA,
        ],
    )
    def gather_kernel(wu_hbm, wi_hbm, xu_hbm, xi_hbm, ou_hbm, oi_hbm,
                      idxu_v, idxi_v, rows0, rows1, g0, g1, w0, w1):
        wid = lax.axis_index("s") * _NC + lax.axis_index("c")
        base = wid * b_per_w
        pltpu.sync_copy(xu_hbm.at[pl.ds(start + base, b_per_w)], idxu_v)
        pltpu.sync_copy(xi_hbm.at[pl.ds(start + base, b_per_w)], idxi_v)

        bufs = (rows0, rows1)
        gsem = (g0, g1)
        wsem = (w0, w1)
        work = [(idxu_v, wu_hbm, ou_hbm, k) for k in range(n_ch)]
        work += [(idxi_v, wi_hbm, oi_hbm, k) for k in range(n_ch)]

        gathers = [None] * len(work)
        writes = [None] * len(work)
        for j, (idx_v, tbl, out, k) in enumerate(work):
            b = j % 2
            if j >= 2:
                writes[j - 2].wait()
            gathers[j] = pltpu.async_copy(
                tbl.at[idx_v.at[pl.ds(k * ch, ch)]], bufs[b], gsem[b])
            if j >= 1:
                pj, (_, _, pout, pk) = j - 1, work[j - 1]
                gathers[pj].wait()
                writes[pj] = pltpu.async_copy(
                    bufs[pj % 2], pout.at[pl.ds(base + pk * ch, ch)],
                    wsem[pj % 2])
        last = len(work) - 1
        _, _, lout, lk = work[last]
        gathers[last].wait()
        writes[last] = pltpu.async_copy(
            bufs[last % 2], lout.at[pl.ds(base + lk * ch, ch)],
            wsem[last % 2])
        writes[last - 1].wait()
        writes[last].wait()

    return gather_kernel(Wu, Wi, x_u, x_i)


def _mlp_body(huA_ref, huB_ref, hiA_ref, hiB_ref, w1_ref, b1_ref, w2_ref,
              b2_ref, wo_ref, bo_ref, o_ref):
    w1 = w1_ref[...]
    dn = (((1,), (1,)), ((), ()))

    def head(hu, hi):
        a = lax.dot_general(hu, w1[:, :_D], dn,
                            preferred_element_type=jnp.float32)
        a = a + lax.dot_general(hi, w1[:, _D:], dn,
                                preferred_element_type=jnp.float32)
        a = a + b1_ref[...]
        a = jnp.where(a >= 0, a, 0.01 * a)
        b = lax.dot_general(a, w2_ref[...], dn,
                            preferred_element_type=jnp.float32)
        b = b + b2_ref[...]
        b = jnp.where(b >= 0, b, 0.01 * b)
        return lax.dot_general(wo_ref[...], b, dn,
                               preferred_element_type=jnp.float32)

    oA = head(huA_ref[...], hiA_ref[...])
    oB = head(huB_ref[...], hiB_ref[...])
    o_ref[...] = jnp.concatenate([oA, oB], axis=1) + bo_ref[...]


def _mlp(hu, hi, W1, b1, W2, b2, Wo, bo, block_rows=1024):
    B = hu.shape[0]
    half = block_rows // 2
    hspec_a = pl.BlockSpec((half, _D), lambda i: (2 * i, 0))
    hspec_b = pl.BlockSpec((half, _D), lambda i: (2 * i + 1, 0))
    return pl.pallas_call(
        _mlp_body,
        grid=(B // block_rows,),
        in_specs=[
            hspec_a,
            hspec_b,
            hspec_a,
            hspec_b,
            pl.BlockSpec((2 * _D, 2 * _D), lambda i: (0, 0)),
            pl.BlockSpec((1, 2 * _D), lambda i: (0, 0)),
            pl.BlockSpec((2 * _D, 2 * _D), lambda i: (0, 0)),
            pl.BlockSpec((1, 2 * _D), lambda i: (0, 0)),
            pl.BlockSpec((1, 2 * _D), lambda i: (0, 0)),
            pl.BlockSpec((1, 1), lambda i: (0, 0)),
        ],
        out_specs=pl.BlockSpec((1, block_rows), lambda i: (0, i)),
        out_shape=jax.ShapeDtypeStruct((1, B), jnp.float32),
    )(hu, hu, hi, hi, W1, b1.reshape(1, -1), W2, b2.reshape(1, -1), Wo,
      bo.reshape(1, 1))


def kernel(x_u, x_i, Wu, Wi, W1, b1, W2, b2, Wo, bo):
    B = x_u.shape[0]
    x_u = x_u.astype(jnp.int32)
    x_i = x_i.astype(jnp.int32)
    h = B // 2
    g0 = _sc_gather_pair(Wu, Wi, x_u, x_i, 0, h)
    g1 = _sc_gather_pair(Wu, Wi, x_u, x_i, h, B - h)
    o0 = _mlp(g0[0], g0[1], W1, b1, W2, b2, Wo, bo, block_rows=4096)
    o1 = _mlp(g1[0], g1[1], W1, b1, W2, b2, Wo, bo, block_rows=4096)
    return jnp.concatenate([o0, o1], axis=1).reshape(B, 1)
